# baseline (speedup 1.0000x reference)
import functools

import jax
import jax.numpy as jnp
from jax import lax
from jax.experimental import pallas as pl
from jax.experimental.pallas import tpu as pltpu
from jax.experimental.pallas import tpu_sc as plsc

N = 10000
E = 640000
F = 128
NUM_HEADS = 8
HID = 16
ROWS_PER_HEAD = N // NUM_HEADS

NC = 2
NS = 16
EDGES_PER_TILE = E // NS
DEG_N = 10240
IDXCH = 4000
K = 88
NCH = EDGES_PER_TILE // K
KT = EDGES_PER_TILE - NCH * K
PN = 10240


def _make_degree_kernel():
    mesh = plsc.VectorSubcoreMesh(core_axis_name="c", subcore_axis_name="s")
    DN = DEG_N
    SEG = DN // NS

    @functools.partial(
        pl.kernel,
        out_type=[
            jax.ShapeDtypeStruct((DN,), jnp.float32),
            jax.ShapeDtypeStruct((DN,), jnp.float32),
        ],
        mesh=mesh,
        compiler_params=pltpu.CompilerParams(needs_layout_passes=False),
        scratch_types=[
            pltpu.VMEM((IDXCH,), jnp.int32),
            pltpu.VMEM((IDXCH,), jnp.int32),
            pltpu.VMEM((DN,), jnp.float32),
            pltpu.VMEM((DN,), jnp.float32),
            pltpu.VMEM((SEG,), jnp.float32),
            pltpu.VMEM_SHARED((NS * DN,), jnp.float32),
            pltpu.SemaphoreType.DMA((2,)),
        ],
    )
    def degree_kernel(dst1_hbm, dst2_hbm, zeros_hbm,
                      deg1_hbm, deg2_hbm,
                      idx_v0, idx_v1, degp, rbuf, outbuf, deg_all, sem_i):
        c = lax.axis_index("c")
        s = lax.axis_index("s")
        ones16 = jnp.full((16,), 1.0, dtype=jnp.float32)

        def run(dst_hbm, out_hbm):
            base = s * EDGES_PER_TILE
            idx_bufs = [idx_v0, idx_v1]
            NCHD = EDGES_PER_TILE // IDXCH

            def load_idx(k, p):
                off = pl.multiple_of(base + k * IDXCH, 8)
                pltpu.async_copy(dst_hbm.at[pl.ds(off, IDXCH)], idx_bufs[p],
                                 sem_i.at[p])

            def wait_idx(p):
                pltpu.make_async_copy(dst_hbm.at[pl.ds(0, IDXCH)],
                                      idx_bufs[p], sem_i.at[p]).wait()

            load_idx(0, 0)
            pltpu.sync_copy(zeros_hbm, degp)

            def process(p):
                def grp(j, _):
                    for u in range(5):
                        iv = idx_bufs[p][pl.ds(j * 80 + u * 16, 16)]
                        plsc.addupdate_scatter(degp, [iv], ones16)
                    return 0

                lax.fori_loop(0, IDXCH // 80, grp, 0)

            load_idx(1, 1)

            def chunk(k, _):
                for p in range(2):
                    kk = 2 * k + p
                    wait_idx(p)
                    process(p)
                    load_idx(kk + 2, p)
                return 0

            lax.fori_loop(0, NCHD // 2 - 1, chunk, 0)
            for p in range(2):
                wait_idx(p)
                process(p)

            pltpu.sync_copy(degp, deg_all.at[pl.ds(s * DN, DN)])
            plsc.subcore_barrier()
            for r in range(NS):
                pltpu.async_copy(deg_all.at[pl.ds(r * DN + s * SEG, SEG)],
                                 rbuf.at[pl.ds(r * SEG, SEG)], sem_i.at[0])
            for r in range(NS):
                pltpu.make_async_copy(deg_all.at[pl.ds(r * DN + s * SEG, SEG)],
                                      rbuf.at[pl.ds(r * SEG, SEG)],
                                      sem_i.at[0]).wait()
            for j in range(SEG // 16):
                acc = rbuf[pl.ds(j * 16, 16)]
                for r in range(1, NS):
                    acc = acc + rbuf[pl.ds(r * SEG + j * 16, 16)]
                outbuf[pl.ds(j * 16, 16)] = acc
            pltpu.sync_copy(outbuf, out_hbm.at[pl.ds(s * SEG, SEG)])

        @pl.when(c == 0)
        def _():
            run(dst1_hbm, deg1_hbm)

        @pl.when(c == 1)
        def _():
            run(dst2_hbm, deg2_hbm)

    return degree_kernel


def _make_propagate_kernel():
    mesh = plsc.VectorSubcoreMesh(core_axis_name="c", subcore_axis_name="s")

    NB = 4

    @functools.partial(
        pl.kernel,
        out_type=[
            jax.ShapeDtypeStruct((PN, F), jnp.float32),
            jax.ShapeDtypeStruct((PN, F), jnp.float32),
        ],
        mesh=mesh,
        compiler_params=pltpu.CompilerParams(needs_layout_passes=False),
        scratch_types=(
            [pltpu.VMEM((K,), jnp.int32) for _ in range(NB)]
            + [pltpu.VMEM((K,), jnp.int32) for _ in range(NB)]
            + [pltpu.VMEM((K, F), jnp.float32) for _ in range(NB)]
            + [
                pltpu.VMEM((KT,), jnp.int32),
                pltpu.VMEM((KT,), jnp.int32),
                pltpu.VMEM_SHARED((PN, F), jnp.float32),
                pltpu.SemaphoreType.DMA((NB,)),
                pltpu.SemaphoreType.DMA((NB,)),
                pltpu.SemaphoreType.DMA((NB,)),
                pltpu.SemaphoreType.DMA((NB,)),
                pltpu.SemaphoreType.DMA,
            ]
        ),
    )
    def prop_kernel(t1_hbm, t2_hbm, src1_hbm, dst1_hbm, src2_hbm, dst2_hbm,
                    zeros_hbm, agg1_hbm, agg2_hbm,
                    si0, si1, si2, si3, di0, di1, di2, di3,
                    rw0, rw1, rw2, rw3, sidx_t, didx_t, agg_sh,
                    sem_is, sem_id, sem_g, sem_s, sem_t):
        c = lax.axis_index("c")
        s = lax.axis_index("s")
        sidx = [si0, si1, si2, si3]
        didx = [di0, di1, di2, di3]
        rows = [rw0, rw1, rw2, rw3]

        def run(table_hbm, src_hbm, dst_hbm, out_hbm):
            zslice = rw0.at[pl.ds(0, 80)]
            pltpu.sync_copy(zeros_hbm, zslice)
            for r in range(8):
                pltpu.async_copy(zslice,
                                 agg_sh.at[pl.ds(s * 640 + r * 80, 80)], sem_t)
            for r in range(8):
                pltpu.make_async_copy(
                    zslice, agg_sh.at[pl.ds(s * 640 + r * 80, 80)],
                    sem_t).wait()
            plsc.subcore_barrier()

            base = s * EDGES_PER_TILE

            def start_idx(ci, q):
                off = pl.multiple_of(base + ci * K, 8)
                pltpu.async_copy(src_hbm.at[pl.ds(off, K)], sidx[q],
                                 sem_is.at[q])
                pltpu.async_copy(dst_hbm.at[pl.ds(off, K)], didx[q],
                                 sem_id.at[q])

            def wait_idx(q):
                pltpu.make_async_copy(src_hbm.at[pl.ds(0, K)], sidx[q],
                                      sem_is.at[q]).wait()
                pltpu.make_async_copy(dst_hbm.at[pl.ds(0, K)], didx[q],
                                      sem_id.at[q]).wait()

            def start_gather(q):
                pltpu.async_copy(table_hbm.at[sidx[q]], rows[q], sem_g.at[q])

            def wait_gather(q):
                pltpu.make_async_copy(table_hbm.at[sidx[q]], rows[q],
                                      sem_g.at[q]).wait()

            def start_scatter(q):
                pltpu.async_copy(rows[q], agg_sh.at[didx[q]], sem_s.at[q],
                                 add=True)

            def wait_scatter(q):
                pltpu.make_async_copy(rows[q], agg_sh.at[didx[q]],
                                      sem_s.at[q]).wait()

            def do_chunk(ci, q, do_sw, do_si, do_g):
                wait_gather(q)
                start_scatter(q)
                if do_sw:
                    wait_scatter((q + 3) % NB)
                if do_si:
                    start_idx(ci + 3, (q + 3) % NB)
                if do_g:
                    wait_idx((q + 2) % NB)
                    start_gather((q + 2) % NB)

            start_idx(0, 0)
            start_idx(1, 1)
            start_idx(2, 2)
            wait_idx(0)
            start_gather(0)
            wait_idx(1)
            start_gather(1)

            do_chunk(0, 0, False, True, True)
            do_chunk(1, 1, True, True, True)
            do_chunk(2, 2, True, True, True)
            do_chunk(3, 3, True, True, True)

            def group(g, _):
                ci0 = g * NB
                for qq in range(NB):
                    do_chunk(ci0 + qq, qq, True, True, True)
                return 0

            lax.fori_loop(1, NCH // NB - 1, group, 0)

            do_chunk(NCH - 6, 0, True, True, True)
            do_chunk(NCH - 5, 1, True, True, True)
            do_chunk(NCH - 4, 2, True, True, True)
            do_chunk(NCH - 3, 3, True, False, True)
            do_chunk(NCH - 2, 0, True, False, False)
            do_chunk(NCH - 1, 1, True, False, False)
            wait_scatter(1)

            off_t = pl.multiple_of(base + NCH * K, 8)
            tslice = rw0.at[pl.ds(0, KT)]
            pltpu.sync_copy(src_hbm.at[pl.ds(off_t, KT)], sidx_t)
            pltpu.sync_copy(dst_hbm.at[pl.ds(off_t, KT)], didx_t)
            pltpu.async_copy(table_hbm.at[sidx_t], tslice, sem_t).wait()
            pltpu.sync_copy(tslice, agg_sh.at[didx_t], add=True)

            plsc.subcore_barrier()

            fbufs = [rows[i].at[pl.ds(0, 80)] for i in range(4)]

            def fl_read(r, p):
                pltpu.async_copy(agg_sh.at[pl.ds(s * 640 + r * 80, 80)],
                                 fbufs[p], sem_g.at[p])

            def fl_read_wait(r, p):
                pltpu.make_async_copy(agg_sh.at[pl.ds(s * 640 + r * 80, 80)],
                                      fbufs[p], sem_g.at[p]).wait()

            def fl_write(r, p):
                pltpu.async_copy(fbufs[p],
                                 out_hbm.at[pl.ds(s * 640 + r * 80, 80)],
                                 sem_s.at[p])

            def fl_write_wait(r, p):
                pltpu.make_async_copy(fbufs[p],
                                      out_hbm.at[pl.ds(s * 640 + r * 80, 80)],
                                      sem_s.at[p]).wait()

            fl_read(0, 0)
            fl_read(1, 1)
            for r in range(8):
                p = r % 4
                fl_read_wait(r, p)
                fl_write(r, p)
                if r + 2 < 8:
                    if r >= 2:
                        fl_write_wait(r - 2, (r + 2) % 4)
                    fl_read(r + 2, (r + 2) % 4)
            for r in range(4, 8):
                fl_write_wait(r, r % 4)

        @pl.when(c == 0)
        def _():
            run(t1_hbm, src1_hbm, dst1_hbm, agg1_hbm)

        @pl.when(c == 1)
        def _():
            run(t2_hbm, src2_hbm, dst2_hbm, agg2_hbm)

    return prop_kernel


BLK = 2000
GRID = N // BLK


def _selector():
    j = lax.broadcasted_iota(jnp.int32, (F, NUM_HEADS), 0)
    cc = lax.broadcasted_iota(jnp.int32, (F, NUM_HEADS), 1)
    return (j // HID == cc).astype(jnp.float32)


def _selector_t():
    cc = lax.broadcasted_iota(jnp.int32, (NUM_HEADS, F), 0)
    j = lax.broadcasted_iota(jnp.int32, (NUM_HEADS, F), 1)
    return (j // HID == cc).astype(jnp.float32)


def _head_onehot(bidx):
    rid = bidx * BLK + lax.broadcasted_iota(jnp.int32, (BLK, NUM_HEADS), 0)
    head = rid // ROWS_PER_HEAD
    hsel = head == lax.broadcasted_iota(jnp.int32, (BLK, NUM_HEADS), 1)
    return hsel.astype(jnp.float32)


def _prep_body(h_ref, wlt_ref, bl_ref, d1_ref, d2_ref, alt_ref,
               t1_ref, t2_ref, n1_ref, n2_ref, ai_ref):
    hh = jnp.dot(h_ref[...], wlt_ref[...],
                 preferred_element_type=jnp.float32) + bl_ref[...]
    d1 = d1_ref[...]
    d2 = d2_ref[...]
    n1 = jnp.where(d1 > 0, lax.rsqrt(jnp.maximum(d1, 1.0)), 0.0)
    n2 = jnp.where(d2 > 0, lax.rsqrt(jnp.maximum(d2, 1.0)), 0.0)
    t1_ref[...] = hh * n1
    t2_ref[...] = hh * n2
    n1_ref[...] = n1
    n2_ref[...] = n2
    airow = jnp.dot(_head_onehot(pl.program_id(0)), alt_ref[...],
                    preferred_element_type=jnp.float32)
    ai_ref[...] = jnp.dot(hh * airow, _selector(),
                          preferred_element_type=jnp.float32)


def _tc_prep(h, wlt, bl, deg1, deg2, alt):
    full = lambda shape: pl.BlockSpec(shape, lambda i: (0, 0))
    row = lambda shape: pl.BlockSpec(shape, lambda i: (i, 0))
    return pl.pallas_call(
        _prep_body,
        grid=(GRID,),
        in_specs=[row((BLK, F)), full((F, F)), full((1, F)),
                  row((BLK, 1)), row((BLK, 1)), full((NUM_HEADS, F))],
        out_specs=[row((BLK, F)), row((BLK, F)),
                   row((BLK, 1)), row((BLK, 1)), row((BLK, NUM_HEADS))],
        out_shape=[
            jax.ShapeDtypeStruct((N, F), jnp.float32),
            jax.ShapeDtypeStruct((N, F), jnp.float32),
            jax.ShapeDtypeStruct((N, 1), jnp.float32),
            jax.ShapeDtypeStruct((N, 1), jnp.float32),
            jax.ShapeDtypeStruct((N, NUM_HEADS), jnp.float32),
        ],
    )(h, wlt, bl, deg1, deg2, alt)


def _combine_body(agg1_ref, agg2_ref, n1_ref, n2_ref, ai_ref, art_ref,
                  wfct_ref, bfc_ref, out_ref):
    h1 = agg1_ref[...] * n1_ref[...]
    h2 = agg2_ref[...] * n2_ref[...]
    arrow = jnp.dot(_head_onehot(pl.program_id(0)), art_ref[...],
                    preferred_element_type=jnp.float32)
    sel = _selector()
    aj1 = jnp.dot(h1 * arrow, sel, preferred_element_type=jnp.float32)
    aj2 = jnp.dot(h2 * arrow, sel, preferred_element_type=jnp.float32)
    ai = ai_ref[...]
    x1 = ai + aj1
    x2 = ai + aj2
    a1 = jnp.clip(jnp.exp(jnp.where(x1 >= 0, x1, 0.2 * x1)), -10.0, 10.0)
    a2 = jnp.clip(jnp.exp(jnp.where(x2 >= 0, x2, 0.2 * x2)), -10.0, 10.0)
    tot = a1 + a2
    al1 = a1 / tot
    al2 = a2 / tot
    st = _selector_t()
    hout = (jnp.dot(al1, st, preferred_element_type=jnp.float32) * h1
            + jnp.dot(al2, st, preferred_element_type=jnp.float32) * h2)
    out_ref[...] = jnp.dot(hout, wfct_ref[...],
                           preferred_element_type=jnp.float32) + bfc_ref[...]


def _tc_combine(agg1, agg2, n1, n2, ai, art, wfct, bfc):
    full = lambda shape: pl.BlockSpec(shape, lambda i: (0, 0))
    row = lambda shape: pl.BlockSpec(shape, lambda i: (i, 0))
    return pl.pallas_call(
        _combine_body,
        grid=(GRID,),
        in_specs=[row((BLK, F)), row((BLK, F)), row((BLK, 1)), row((BLK, 1)),
                  row((BLK, NUM_HEADS)), full((NUM_HEADS, F)),
                  full((F, F)), full((1, F))],
        out_specs=row((BLK, F)),
        out_shape=jax.ShapeDtypeStruct((N, F), jnp.float32),
    )(agg1, agg2, n1, n2, ai, art, wfct, bfc)


def kernel(h, edge_index1, edge_index2, W_lin, b_lin, al, ar, W_fc, b_fc):
    src1, dst1 = edge_index1[0], edge_index1[1]
    src2, dst2 = edge_index2[0], edge_index2[1]

    zeros_deg = jnp.zeros((DEG_N,), jnp.float32)
    deg1_flat, deg2_flat = _make_degree_kernel()(dst1, dst2, zeros_deg)
    deg1 = deg1_flat[:N].reshape(N, 1)
    deg2 = deg2_flat[:N].reshape(N, 1)

    wlt = W_lin.T
    bl = b_lin.reshape(1, F)
    alt = jnp.tile(al[:, :, 0], (1, NUM_HEADS))
    art = jnp.tile(ar[:, :, 0], (1, NUM_HEADS))

    t1, t2, n1, n2, ai = _tc_prep(h, wlt, bl, deg1, deg2, alt)

    zeros_rows = jnp.zeros((80, F), jnp.float32)
    agg1, agg2 = _make_propagate_kernel()(t1, t2, src1, dst1, src2, dst2,
                                          zeros_rows)

    return _tc_combine(agg1[:N], agg2[:N], n1, n2, ai, art, W_fc.T,
                       b_fc.reshape(1, F))


# speedup vs baseline: 1.0448x; 1.0448x over previous
import functools

import jax
import jax.numpy as jnp
from jax import lax
from jax.experimental import pallas as pl
from jax.experimental.pallas import tpu as pltpu
from jax.experimental.pallas import tpu_sc as plsc

N = 10000
E = 640000
F = 128
NUM_HEADS = 8
HID = 16
ROWS_PER_HEAD = N // NUM_HEADS

NC = 2
NS = 16
EDGES_PER_TILE = E // NS
DEG_N = 10240
IDXCH = 4000
K = 88
NCH = EDGES_PER_TILE // K
KT = EDGES_PER_TILE - NCH * K
PN = 10240


def _make_degree_kernel():
    mesh = plsc.VectorSubcoreMesh(core_axis_name="c", subcore_axis_name="s")
    DN = DEG_N
    SEG = DN // NS

    @functools.partial(
        pl.kernel,
        out_type=[
            jax.ShapeDtypeStruct((DN,), jnp.float32),
            jax.ShapeDtypeStruct((DN,), jnp.float32),
        ],
        mesh=mesh,
        compiler_params=pltpu.CompilerParams(needs_layout_passes=False),
        scratch_types=[
            pltpu.VMEM((IDXCH,), jnp.int32),
            pltpu.VMEM((IDXCH,), jnp.int32),
            pltpu.VMEM((DN,), jnp.float32),
            pltpu.VMEM((DN,), jnp.float32),
            pltpu.VMEM((SEG,), jnp.float32),
            pltpu.VMEM_SHARED((NS * DN,), jnp.float32),
            pltpu.SemaphoreType.DMA((2,)),
        ],
    )
    def degree_kernel(ei1_hbm, ei2_hbm, zeros_hbm,
                      deg1_hbm, deg2_hbm,
                      idx_v0, idx_v1, degp, rbuf, outbuf, deg_all, sem_i):
        c = lax.axis_index("c")
        s = lax.axis_index("s")
        ones16 = jnp.full((16,), 1.0, dtype=jnp.float32)

        def run(dst_hbm, out_hbm):
            base = E + s * EDGES_PER_TILE
            idx_bufs = [idx_v0, idx_v1]
            NCHD = EDGES_PER_TILE // IDXCH

            def load_idx(k, p):
                off = pl.multiple_of(base + k * IDXCH, 8)
                pltpu.async_copy(dst_hbm.at[pl.ds(off, IDXCH)], idx_bufs[p],
                                 sem_i.at[p])

            def wait_idx(p):
                pltpu.make_async_copy(dst_hbm.at[pl.ds(0, IDXCH)],
                                      idx_bufs[p], sem_i.at[p]).wait()

            load_idx(0, 0)
            pltpu.sync_copy(zeros_hbm, degp)

            def process(p):
                def grp(j, _):
                    for u in range(5):
                        iv = idx_bufs[p][pl.ds(j * 80 + u * 16, 16)]
                        plsc.addupdate_scatter(degp, [iv], ones16)
                    return 0

                lax.fori_loop(0, IDXCH // 80, grp, 0)

            load_idx(1, 1)

            def chunk(k, _):
                for p in range(2):
                    kk = 2 * k + p
                    wait_idx(p)
                    process(p)
                    load_idx(kk + 2, p)
                return 0

            lax.fori_loop(0, NCHD // 2 - 1, chunk, 0)
            for p in range(2):
                wait_idx(p)
                process(p)

            pltpu.sync_copy(degp, deg_all.at[pl.ds(s * DN, DN)])
            plsc.subcore_barrier()
            for r in range(NS):
                pltpu.async_copy(deg_all.at[pl.ds(r * DN + s * SEG, SEG)],
                                 rbuf.at[pl.ds(r * SEG, SEG)], sem_i.at[0])
            for r in range(NS):
                pltpu.make_async_copy(deg_all.at[pl.ds(r * DN + s * SEG, SEG)],
                                      rbuf.at[pl.ds(r * SEG, SEG)],
                                      sem_i.at[0]).wait()
            for j in range(SEG // 16):
                acc = rbuf[pl.ds(j * 16, 16)]
                for r in range(1, NS):
                    acc = acc + rbuf[pl.ds(r * SEG + j * 16, 16)]
                outbuf[pl.ds(j * 16, 16)] = acc
            pltpu.sync_copy(outbuf, out_hbm.at[pl.ds(s * SEG, SEG)])

        @pl.when(c == 0)
        def _():
            run(ei1_hbm, deg1_hbm)

        @pl.when(c == 1)
        def _():
            run(ei2_hbm, deg2_hbm)

    return degree_kernel


def _make_propagate_kernel():
    mesh = plsc.VectorSubcoreMesh(core_axis_name="c", subcore_axis_name="s")

    NB = 4

    @functools.partial(
        pl.kernel,
        out_type=[
            jax.ShapeDtypeStruct((PN, F), jnp.float32),
            jax.ShapeDtypeStruct((PN, F), jnp.float32),
        ],
        mesh=mesh,
        compiler_params=pltpu.CompilerParams(needs_layout_passes=False),
        scratch_types=(
            [pltpu.VMEM((K,), jnp.int32) for _ in range(NB)]
            + [pltpu.VMEM((K,), jnp.int32) for _ in range(NB)]
            + [pltpu.VMEM((K, F), jnp.float32) for _ in range(NB)]
            + [
                pltpu.VMEM((KT,), jnp.int32),
                pltpu.VMEM((KT,), jnp.int32),
                pltpu.VMEM_SHARED((PN, F), jnp.float32),
                pltpu.SemaphoreType.DMA((NB,)),
                pltpu.SemaphoreType.DMA((NB,)),
                pltpu.SemaphoreType.DMA((NB,)),
                pltpu.SemaphoreType.DMA((NB,)),
                pltpu.SemaphoreType.DMA,
            ]
        ),
    )
    def prop_kernel(t1_hbm, t2_hbm, ei1_hbm, ei2_hbm,
                    zeros_hbm, agg1_hbm, agg2_hbm,
                    si0, si1, si2, si3, di0, di1, di2, di3,
                    rw0, rw1, rw2, rw3, sidx_t, didx_t, agg_sh,
                    sem_is, sem_id, sem_g, sem_s, sem_t):
        c = lax.axis_index("c")
        s = lax.axis_index("s")
        sidx = [si0, si1, si2, si3]
        didx = [di0, di1, di2, di3]
        rows = [rw0, rw1, rw2, rw3]

        def run(table_hbm, ei_hbm, out_hbm):
            src_hbm = ei_hbm
            dst_hbm = ei_hbm
            zslice = rw0.at[pl.ds(0, 80)]
            pltpu.sync_copy(zeros_hbm, zslice)
            for r in range(8):
                pltpu.async_copy(zslice,
                                 agg_sh.at[pl.ds(s * 640 + r * 80, 80)], sem_t)
            for r in range(8):
                pltpu.make_async_copy(
                    zslice, agg_sh.at[pl.ds(s * 640 + r * 80, 80)],
                    sem_t).wait()
            plsc.subcore_barrier()

            base = s * EDGES_PER_TILE

            def start_idx(ci, q):
                off = pl.multiple_of(base + ci * K, 8)
                offd = pl.multiple_of(E + base + ci * K, 8)
                pltpu.async_copy(src_hbm.at[pl.ds(off, K)], sidx[q],
                                 sem_is.at[q])
                pltpu.async_copy(dst_hbm.at[pl.ds(offd, K)], didx[q],
                                 sem_id.at[q])

            def wait_idx(q):
                pltpu.make_async_copy(src_hbm.at[pl.ds(0, K)], sidx[q],
                                      sem_is.at[q]).wait()
                pltpu.make_async_copy(dst_hbm.at[pl.ds(0, K)], didx[q],
                                      sem_id.at[q]).wait()

            def start_gather(q):
                pltpu.async_copy(table_hbm.at[sidx[q]], rows[q], sem_g.at[q])

            def wait_gather(q):
                pltpu.make_async_copy(table_hbm.at[sidx[q]], rows[q],
                                      sem_g.at[q]).wait()

            def start_scatter(q):
                pltpu.async_copy(rows[q], agg_sh.at[didx[q]], sem_s.at[q],
                                 add=True)

            def wait_scatter(q):
                pltpu.make_async_copy(rows[q], agg_sh.at[didx[q]],
                                      sem_s.at[q]).wait()

            def do_chunk(ci, q, do_sw, do_si, do_g):
                wait_gather(q)
                start_scatter(q)
                if do_sw:
                    wait_scatter((q + 3) % NB)
                if do_si:
                    start_idx(ci + 3, (q + 3) % NB)
                if do_g:
                    wait_idx((q + 2) % NB)
                    start_gather((q + 2) % NB)

            start_idx(0, 0)
            start_idx(1, 1)
            start_idx(2, 2)
            wait_idx(0)
            start_gather(0)
            wait_idx(1)
            start_gather(1)

            do_chunk(0, 0, False, True, True)
            do_chunk(1, 1, True, True, True)
            do_chunk(2, 2, True, True, True)
            do_chunk(3, 3, True, True, True)

            def group(g, _):
                ci0 = g * NB
                for qq in range(NB):
                    do_chunk(ci0 + qq, qq, True, True, True)
                return 0

            lax.fori_loop(1, NCH // NB - 1, group, 0)

            do_chunk(NCH - 6, 0, True, True, True)
            do_chunk(NCH - 5, 1, True, True, True)
            do_chunk(NCH - 4, 2, True, True, True)
            do_chunk(NCH - 3, 3, True, False, True)
            do_chunk(NCH - 2, 0, True, False, False)
            do_chunk(NCH - 1, 1, True, False, False)
            wait_scatter(1)

            off_t = pl.multiple_of(base + NCH * K, 8)
            offd_t = pl.multiple_of(E + base + NCH * K, 8)
            tslice = rw0.at[pl.ds(0, KT)]
            pltpu.sync_copy(src_hbm.at[pl.ds(off_t, KT)], sidx_t)
            pltpu.sync_copy(dst_hbm.at[pl.ds(offd_t, KT)], didx_t)
            pltpu.async_copy(table_hbm.at[sidx_t], tslice, sem_t).wait()
            pltpu.sync_copy(tslice, agg_sh.at[didx_t], add=True)

            plsc.subcore_barrier()

            fbufs = [rows[i].at[pl.ds(0, 80)] for i in range(4)]

            def fl_read(r, p):
                pltpu.async_copy(agg_sh.at[pl.ds(s * 640 + r * 80, 80)],
                                 fbufs[p], sem_g.at[p])

            def fl_read_wait(r, p):
                pltpu.make_async_copy(agg_sh.at[pl.ds(s * 640 + r * 80, 80)],
                                      fbufs[p], sem_g.at[p]).wait()

            def fl_write(r, p):
                pltpu.async_copy(fbufs[p],
                                 out_hbm.at[pl.ds(s * 640 + r * 80, 80)],
                                 sem_s.at[p])

            def fl_write_wait(r, p):
                pltpu.make_async_copy(fbufs[p],
                                      out_hbm.at[pl.ds(s * 640 + r * 80, 80)],
                                      sem_s.at[p]).wait()

            fl_read(0, 0)
            fl_read(1, 1)
            for r in range(8):
                p = r % 4
                fl_read_wait(r, p)
                fl_write(r, p)
                if r + 2 < 8:
                    if r >= 2:
                        fl_write_wait(r - 2, (r + 2) % 4)
                    fl_read(r + 2, (r + 2) % 4)
            for r in range(4, 8):
                fl_write_wait(r, r % 4)

        @pl.when(c == 0)
        def _():
            run(t1_hbm, ei1_hbm, agg1_hbm)

        @pl.when(c == 1)
        def _():
            run(t2_hbm, ei2_hbm, agg2_hbm)

    return prop_kernel


BLK = 2000
GRID = N // BLK


def _selector():
    j = lax.broadcasted_iota(jnp.int32, (F, NUM_HEADS), 0)
    cc = lax.broadcasted_iota(jnp.int32, (F, NUM_HEADS), 1)
    return (j // HID == cc).astype(jnp.float32)


def _selector_t():
    cc = lax.broadcasted_iota(jnp.int32, (NUM_HEADS, F), 0)
    j = lax.broadcasted_iota(jnp.int32, (NUM_HEADS, F), 1)
    return (j // HID == cc).astype(jnp.float32)


def _head_onehot(bidx):
    rid = bidx * BLK + lax.broadcasted_iota(jnp.int32, (BLK, NUM_HEADS), 0)
    head = rid // ROWS_PER_HEAD
    hsel = head == lax.broadcasted_iota(jnp.int32, (BLK, NUM_HEADS), 1)
    return hsel.astype(jnp.float32)


def _prep_body(h_ref, wlt_ref, bl_ref, d1_ref, d2_ref, alt_ref,
               t1_ref, t2_ref, n1_ref, n2_ref, ai_ref):
    hh = jnp.dot(h_ref[...], wlt_ref[...],
                 preferred_element_type=jnp.float32) + bl_ref[...]
    d1 = d1_ref[...]
    d2 = d2_ref[...]
    n1 = jnp.where(d1 > 0, lax.rsqrt(jnp.maximum(d1, 1.0)), 0.0)
    n2 = jnp.where(d2 > 0, lax.rsqrt(jnp.maximum(d2, 1.0)), 0.0)
    t1_ref[...] = hh * n1
    t2_ref[...] = hh * n2
    n1_ref[...] = n1
    n2_ref[...] = n2
    airow = jnp.dot(_head_onehot(pl.program_id(0)), alt_ref[...],
                    preferred_element_type=jnp.float32)
    ai_ref[...] = jnp.dot(hh * airow, _selector(),
                          preferred_element_type=jnp.float32)


def _tc_prep(h, wlt, bl, deg1, deg2, alt):
    full = lambda shape: pl.BlockSpec(shape, lambda i: (0, 0))
    row = lambda shape: pl.BlockSpec(shape, lambda i: (i, 0))
    return pl.pallas_call(
        _prep_body,
        grid=(GRID,),
        in_specs=[row((BLK, F)), full((F, F)), full((1, F)),
                  row((BLK, 1)), row((BLK, 1)), full((NUM_HEADS, F))],
        out_specs=[row((BLK, F)), row((BLK, F)),
                   row((BLK, 1)), row((BLK, 1)), row((BLK, NUM_HEADS))],
        out_shape=[
            jax.ShapeDtypeStruct((N, F), jnp.float32),
            jax.ShapeDtypeStruct((N, F), jnp.float32),
            jax.ShapeDtypeStruct((N, 1), jnp.float32),
            jax.ShapeDtypeStruct((N, 1), jnp.float32),
            jax.ShapeDtypeStruct((N, NUM_HEADS), jnp.float32),
        ],
    )(h, wlt, bl, deg1, deg2, alt)


def _combine_body(agg1_ref, agg2_ref, n1_ref, n2_ref, ai_ref, art_ref,
                  wfct_ref, bfc_ref, out_ref):
    h1 = agg1_ref[...] * n1_ref[...]
    h2 = agg2_ref[...] * n2_ref[...]
    arrow = jnp.dot(_head_onehot(pl.program_id(0)), art_ref[...],
                    preferred_element_type=jnp.float32)
    sel = _selector()
    aj1 = jnp.dot(h1 * arrow, sel, preferred_element_type=jnp.float32)
    aj2 = jnp.dot(h2 * arrow, sel, preferred_element_type=jnp.float32)
    ai = ai_ref[...]
    x1 = ai + aj1
    x2 = ai + aj2
    a1 = jnp.clip(jnp.exp(jnp.where(x1 >= 0, x1, 0.2 * x1)), -10.0, 10.0)
    a2 = jnp.clip(jnp.exp(jnp.where(x2 >= 0, x2, 0.2 * x2)), -10.0, 10.0)
    tot = a1 + a2
    al1 = a1 / tot
    al2 = a2 / tot
    st = _selector_t()
    hout = (jnp.dot(al1, st, preferred_element_type=jnp.float32) * h1
            + jnp.dot(al2, st, preferred_element_type=jnp.float32) * h2)
    out_ref[...] = jnp.dot(hout, wfct_ref[...],
                           preferred_element_type=jnp.float32) + bfc_ref[...]


def _tc_combine(agg1, agg2, n1, n2, ai, art, wfct, bfc):
    full = lambda shape: pl.BlockSpec(shape, lambda i: (0, 0))
    row = lambda shape: pl.BlockSpec(shape, lambda i: (i, 0))
    return pl.pallas_call(
        _combine_body,
        grid=(GRID,),
        in_specs=[row((BLK, F)), row((BLK, F)), row((BLK, 1)), row((BLK, 1)),
                  row((BLK, NUM_HEADS)), full((NUM_HEADS, F)),
                  full((F, F)), full((1, F))],
        out_specs=row((BLK, F)),
        out_shape=jax.ShapeDtypeStruct((N, F), jnp.float32),
    )(agg1, agg2, n1, n2, ai, art, wfct, bfc)


def kernel(h, edge_index1, edge_index2, W_lin, b_lin, al, ar, W_fc, b_fc):
    ei1 = edge_index1.reshape(-1)
    ei2 = edge_index2.reshape(-1)

    zeros_deg = jnp.zeros((DEG_N,), jnp.float32)
    deg1_flat, deg2_flat = _make_degree_kernel()(ei1, ei2, zeros_deg)
    deg1 = deg1_flat.reshape(DEG_N, 1)
    deg2 = deg2_flat.reshape(DEG_N, 1)

    wlt = W_lin.T
    bl = b_lin.reshape(1, F)
    alt = jnp.tile(al[:, :, 0], (1, NUM_HEADS))
    art = jnp.tile(ar[:, :, 0], (1, NUM_HEADS))

    t1, t2, n1, n2, ai = _tc_prep(h, wlt, bl, deg1, deg2, alt)

    zeros_rows = jnp.zeros((80, F), jnp.float32)
    agg1, agg2 = _make_propagate_kernel()(t1, t2, ei1, ei2, zeros_rows)

    return _tc_combine(agg1, agg2, n1, n2, ai, art, W_fc.T,
                       b_fc.reshape(1, F))


# speedup vs baseline: 1.0449x; 1.0001x over previous
import functools

import jax
import jax.numpy as jnp
from jax import lax
from jax.experimental import pallas as pl
from jax.experimental.pallas import tpu as pltpu
from jax.experimental.pallas import tpu_sc as plsc

N = 10000
E = 640000
F = 128
NUM_HEADS = 8
HID = 16
ROWS_PER_HEAD = N // NUM_HEADS

NC = 2
NS = 16
EDGES_PER_TILE = E // NS
DEG_N = 10240
IDXCH = 4000
K = 88
NCH = EDGES_PER_TILE // K
KT = EDGES_PER_TILE - NCH * K
PN = 10240


def _make_degree_kernel():
    mesh = plsc.VectorSubcoreMesh(core_axis_name="c", subcore_axis_name="s")
    DN = DEG_N
    SEG = DN // NS

    @functools.partial(
        pl.kernel,
        out_type=[
            jax.ShapeDtypeStruct((DN,), jnp.float32),
            jax.ShapeDtypeStruct((DN,), jnp.float32),
        ],
        mesh=mesh,
        compiler_params=pltpu.CompilerParams(needs_layout_passes=False),
        scratch_types=[
            pltpu.VMEM((IDXCH,), jnp.int32),
            pltpu.VMEM((IDXCH,), jnp.int32),
            pltpu.VMEM((DN,), jnp.float32),
            pltpu.VMEM((DN,), jnp.float32),
            pltpu.VMEM((SEG,), jnp.float32),
            pltpu.VMEM_SHARED((NS * DN,), jnp.float32),
            pltpu.SemaphoreType.DMA((2,)),
        ],
    )
    def degree_kernel(ei1_hbm, ei2_hbm, zeros_hbm,
                      deg1_hbm, deg2_hbm,
                      idx_v0, idx_v1, degp, rbuf, outbuf, deg_all, sem_i):
        c = lax.axis_index("c")
        s = lax.axis_index("s")
        ones16 = jnp.full((16,), 1.0, dtype=jnp.float32)

        def run(dst_hbm, out_hbm):
            base = E + s * EDGES_PER_TILE
            idx_bufs = [idx_v0, idx_v1]
            NCHD = EDGES_PER_TILE // IDXCH

            def load_idx(k, p):
                off = pl.multiple_of(base + k * IDXCH, 8)
                pltpu.async_copy(dst_hbm.at[pl.ds(off, IDXCH)], idx_bufs[p],
                                 sem_i.at[p])

            def wait_idx(p):
                pltpu.make_async_copy(dst_hbm.at[pl.ds(0, IDXCH)],
                                      idx_bufs[p], sem_i.at[p]).wait()

            load_idx(0, 0)
            pltpu.sync_copy(zeros_hbm, degp)

            def process(p):
                def grp(j, _):
                    for u in range(5):
                        iv = idx_bufs[p][pl.ds(j * 80 + u * 16, 16)]
                        plsc.addupdate_scatter(degp, [iv], ones16)
                    return 0

                lax.fori_loop(0, IDXCH // 80, grp, 0)

            load_idx(1, 1)

            def chunk(k, _):
                for p in range(2):
                    kk = 2 * k + p
                    wait_idx(p)
                    process(p)
                    load_idx(kk + 2, p)
                return 0

            lax.fori_loop(0, NCHD // 2 - 1, chunk, 0)
            for p in range(2):
                wait_idx(p)
                process(p)

            pltpu.sync_copy(degp, deg_all.at[pl.ds(s * DN, DN)])
            plsc.subcore_barrier()
            for r in range(NS):
                pltpu.async_copy(deg_all.at[pl.ds(r * DN + s * SEG, SEG)],
                                 rbuf.at[pl.ds(r * SEG, SEG)], sem_i.at[0])
            for r in range(NS):
                pltpu.make_async_copy(deg_all.at[pl.ds(r * DN + s * SEG, SEG)],
                                      rbuf.at[pl.ds(r * SEG, SEG)],
                                      sem_i.at[0]).wait()
            for j in range(SEG // 16):
                acc = rbuf[pl.ds(j * 16, 16)]
                for r in range(1, NS):
                    acc = acc + rbuf[pl.ds(r * SEG + j * 16, 16)]
                outbuf[pl.ds(j * 16, 16)] = acc
            pltpu.sync_copy(outbuf, out_hbm.at[pl.ds(s * SEG, SEG)])

        @pl.when(c == 0)
        def _():
            run(ei1_hbm, deg1_hbm)

        @pl.when(c == 1)
        def _():
            run(ei2_hbm, deg2_hbm)

    return degree_kernel


def _make_propagate_kernel():
    mesh = plsc.VectorSubcoreMesh(core_axis_name="c", subcore_axis_name="s")

    NB = 4

    @functools.partial(
        pl.kernel,
        out_type=[
            jax.ShapeDtypeStruct((PN, F), jnp.float32),
            jax.ShapeDtypeStruct((PN, F), jnp.float32),
        ],
        mesh=mesh,
        compiler_params=pltpu.CompilerParams(needs_layout_passes=False),
        scratch_types=(
            [pltpu.VMEM((K,), jnp.int32) for _ in range(NB)]
            + [pltpu.VMEM((K,), jnp.int32) for _ in range(NB)]
            + [pltpu.VMEM((K, F), jnp.float32) for _ in range(NB)]
            + [
                pltpu.VMEM((KT,), jnp.int32),
                pltpu.VMEM((KT,), jnp.int32),
                pltpu.VMEM_SHARED((PN, F), jnp.float32),
                pltpu.SemaphoreType.DMA((NB,)),
                pltpu.SemaphoreType.DMA((NB,)),
                pltpu.SemaphoreType.DMA((NB,)),
                pltpu.SemaphoreType.DMA((NB,)),
                pltpu.SemaphoreType.DMA,
            ]
        ),
    )
    def prop_kernel(t1_hbm, t2_hbm, ei1_hbm, ei2_hbm,
                    zeros_hbm, agg1_hbm, agg2_hbm,
                    si0, si1, si2, si3, di0, di1, di2, di3,
                    rw0, rw1, rw2, rw3, sidx_t, didx_t, agg_sh,
                    sem_is, sem_id, sem_g, sem_s, sem_t):
        c = lax.axis_index("c")
        s = lax.axis_index("s")
        sidx = [si0, si1, si2, si3]
        didx = [di0, di1, di2, di3]
        rows = [rw0, rw1, rw2, rw3]

        def run(table_hbm, ei_hbm, out_hbm):
            src_hbm = ei_hbm
            dst_hbm = ei_hbm
            base = s * EDGES_PER_TILE

            def start_idx(ci, q):
                off = pl.multiple_of(base + ci * K, 8)
                offd = pl.multiple_of(E + base + ci * K, 8)
                pltpu.async_copy(src_hbm.at[pl.ds(off, K)], sidx[q],
                                 sem_is.at[q])
                pltpu.async_copy(dst_hbm.at[pl.ds(offd, K)], didx[q],
                                 sem_id.at[q])

            def wait_idx(q):
                pltpu.make_async_copy(src_hbm.at[pl.ds(0, K)], sidx[q],
                                      sem_is.at[q]).wait()
                pltpu.make_async_copy(dst_hbm.at[pl.ds(0, K)], didx[q],
                                      sem_id.at[q]).wait()

            def start_gather(q):
                pltpu.async_copy(table_hbm.at[sidx[q]], rows[q], sem_g.at[q])

            def wait_gather(q):
                pltpu.make_async_copy(table_hbm.at[sidx[q]], rows[q],
                                      sem_g.at[q]).wait()

            def start_scatter(q):
                pltpu.async_copy(rows[q], agg_sh.at[didx[q]], sem_s.at[q],
                                 add=True)

            def wait_scatter(q):
                pltpu.make_async_copy(rows[q], agg_sh.at[didx[q]],
                                      sem_s.at[q]).wait()

            def do_chunk(ci, q, do_sw, do_si, do_g):
                wait_gather(q)
                start_scatter(q)
                if do_sw:
                    wait_scatter((q + 3) % NB)
                if do_si:
                    start_idx(ci + 3, (q + 3) % NB)
                if do_g:
                    wait_idx((q + 2) % NB)
                    start_gather((q + 2) % NB)

            start_idx(0, 0)
            start_idx(1, 1)
            start_idx(2, 2)
            wait_idx(0)
            start_gather(0)
            wait_idx(1)
            start_gather(1)

            zslice = rw3.at[pl.ds(0, 80)]
            pltpu.sync_copy(zeros_hbm, zslice)
            for r in range(8):
                pltpu.async_copy(zslice,
                                 agg_sh.at[pl.ds(s * 640 + r * 80, 80)], sem_t)
            for r in range(8):
                pltpu.make_async_copy(
                    zslice, agg_sh.at[pl.ds(s * 640 + r * 80, 80)],
                    sem_t).wait()
            plsc.subcore_barrier()

            do_chunk(0, 0, False, True, True)
            do_chunk(1, 1, True, True, True)
            do_chunk(2, 2, True, True, True)
            do_chunk(3, 3, True, True, True)

            def group(g, _):
                ci0 = g * NB
                for qq in range(NB):
                    do_chunk(ci0 + qq, qq, True, True, True)
                return 0

            lax.fori_loop(1, NCH // NB - 1, group, 0)

            do_chunk(NCH - 6, 0, True, True, True)
            do_chunk(NCH - 5, 1, True, True, True)
            do_chunk(NCH - 4, 2, True, True, True)
            do_chunk(NCH - 3, 3, True, False, True)
            do_chunk(NCH - 2, 0, True, False, False)
            do_chunk(NCH - 1, 1, True, False, False)
            wait_scatter(1)

            off_t = pl.multiple_of(base + NCH * K, 8)
            offd_t = pl.multiple_of(E + base + NCH * K, 8)
            tslice = rw0.at[pl.ds(0, KT)]
            pltpu.sync_copy(src_hbm.at[pl.ds(off_t, KT)], sidx_t)
            pltpu.sync_copy(dst_hbm.at[pl.ds(offd_t, KT)], didx_t)
            pltpu.async_copy(table_hbm.at[sidx_t], tslice, sem_t).wait()
            pltpu.sync_copy(tslice, agg_sh.at[didx_t], add=True)

            plsc.subcore_barrier()

            fbufs = [rows[i].at[pl.ds(0, 80)] for i in range(4)]

            def fl_read(r, p):
                pltpu.async_copy(agg_sh.at[pl.ds(s * 640 + r * 80, 80)],
                                 fbufs[p], sem_g.at[p])

            def fl_read_wait(r, p):
                pltpu.make_async_copy(agg_sh.at[pl.ds(s * 640 + r * 80, 80)],
                                      fbufs[p], sem_g.at[p]).wait()

            def fl_write(r, p):
                pltpu.async_copy(fbufs[p],
                                 out_hbm.at[pl.ds(s * 640 + r * 80, 80)],
                                 sem_s.at[p])

            def fl_write_wait(r, p):
                pltpu.make_async_copy(fbufs[p],
                                      out_hbm.at[pl.ds(s * 640 + r * 80, 80)],
                                      sem_s.at[p]).wait()

            fl_read(0, 0)
            fl_read(1, 1)
            for r in range(8):
                p = r % 4
                fl_read_wait(r, p)
                fl_write(r, p)
                if r + 2 < 8:
                    if r >= 2:
                        fl_write_wait(r - 2, (r + 2) % 4)
                    fl_read(r + 2, (r + 2) % 4)
            for r in range(4, 8):
                fl_write_wait(r, r % 4)

        @pl.when(c == 0)
        def _():
            run(t1_hbm, ei1_hbm, agg1_hbm)

        @pl.when(c == 1)
        def _():
            run(t2_hbm, ei2_hbm, agg2_hbm)

    return prop_kernel


BLK = 2000
GRID = N // BLK


def _selector():
    j = lax.broadcasted_iota(jnp.int32, (F, NUM_HEADS), 0)
    cc = lax.broadcasted_iota(jnp.int32, (F, NUM_HEADS), 1)
    return (j // HID == cc).astype(jnp.float32)


def _selector_t():
    cc = lax.broadcasted_iota(jnp.int32, (NUM_HEADS, F), 0)
    j = lax.broadcasted_iota(jnp.int32, (NUM_HEADS, F), 1)
    return (j // HID == cc).astype(jnp.float32)


def _head_onehot(bidx):
    rid = bidx * BLK + lax.broadcasted_iota(jnp.int32, (BLK, NUM_HEADS), 0)
    head = rid // ROWS_PER_HEAD
    hsel = head == lax.broadcasted_iota(jnp.int32, (BLK, NUM_HEADS), 1)
    return hsel.astype(jnp.float32)


def _prep_body(h_ref, wlt_ref, bl_ref, d1_ref, d2_ref, alt_ref,
               t1_ref, t2_ref, n1_ref, n2_ref, ai_ref):
    hh = jnp.dot(h_ref[...], wlt_ref[...],
                 preferred_element_type=jnp.float32) + bl_ref[...]
    d1 = d1_ref[...]
    d2 = d2_ref[...]
    n1 = jnp.where(d1 > 0, lax.rsqrt(jnp.maximum(d1, 1.0)), 0.0)
    n2 = jnp.where(d2 > 0, lax.rsqrt(jnp.maximum(d2, 1.0)), 0.0)
    t1_ref[...] = hh * n1
    t2_ref[...] = hh * n2
    n1_ref[...] = n1
    n2_ref[...] = n2
    airow = jnp.dot(_head_onehot(pl.program_id(0)), alt_ref[...],
                    preferred_element_type=jnp.float32)
    ai_ref[...] = jnp.dot(hh * airow, _selector(),
                          preferred_element_type=jnp.float32)


def _tc_prep(h, wlt, bl, deg1, deg2, alt):
    full = lambda shape: pl.BlockSpec(shape, lambda i: (0, 0))
    row = lambda shape: pl.BlockSpec(shape, lambda i: (i, 0))
    return pl.pallas_call(
        _prep_body,
        grid=(GRID,),
        in_specs=[row((BLK, F)), full((F, F)), full((1, F)),
                  row((BLK, 1)), row((BLK, 1)), full((NUM_HEADS, F))],
        out_specs=[row((BLK, F)), row((BLK, F)),
                   row((BLK, 1)), row((BLK, 1)), row((BLK, NUM_HEADS))],
        out_shape=[
            jax.ShapeDtypeStruct((N, F), jnp.float32),
            jax.ShapeDtypeStruct((N, F), jnp.float32),
            jax.ShapeDtypeStruct((N, 1), jnp.float32),
            jax.ShapeDtypeStruct((N, 1), jnp.float32),
            jax.ShapeDtypeStruct((N, NUM_HEADS), jnp.float32),
        ],
    )(h, wlt, bl, deg1, deg2, alt)


def _combine_body(agg1_ref, agg2_ref, n1_ref, n2_ref, ai_ref, art_ref,
                  wfct_ref, bfc_ref, out_ref):
    h1 = agg1_ref[...] * n1_ref[...]
    h2 = agg2_ref[...] * n2_ref[...]
    arrow = jnp.dot(_head_onehot(pl.program_id(0)), art_ref[...],
                    preferred_element_type=jnp.float32)
    sel = _selector()
    aj1 = jnp.dot(h1 * arrow, sel, preferred_element_type=jnp.float32)
    aj2 = jnp.dot(h2 * arrow, sel, preferred_element_type=jnp.float32)
    ai = ai_ref[...]
    x1 = ai + aj1
    x2 = ai + aj2
    a1 = jnp.clip(jnp.exp(jnp.where(x1 >= 0, x1, 0.2 * x1)), -10.0, 10.0)
    a2 = jnp.clip(jnp.exp(jnp.where(x2 >= 0, x2, 0.2 * x2)), -10.0, 10.0)
    tot = a1 + a2
    al1 = a1 / tot
    al2 = a2 / tot
    st = _selector_t()
    hout = (jnp.dot(al1, st, preferred_element_type=jnp.float32) * h1
            + jnp.dot(al2, st, preferred_element_type=jnp.float32) * h2)
    out_ref[...] = jnp.dot(hout, wfct_ref[...],
                           preferred_element_type=jnp.float32) + bfc_ref[...]


def _tc_combine(agg1, agg2, n1, n2, ai, art, wfct, bfc):
    full = lambda shape: pl.BlockSpec(shape, lambda i: (0, 0))
    row = lambda shape: pl.BlockSpec(shape, lambda i: (i, 0))
    return pl.pallas_call(
        _combine_body,
        grid=(GRID,),
        in_specs=[row((BLK, F)), row((BLK, F)), row((BLK, 1)), row((BLK, 1)),
                  row((BLK, NUM_HEADS)), full((NUM_HEADS, F)),
                  full((F, F)), full((1, F))],
        out_specs=row((BLK, F)),
        out_shape=jax.ShapeDtypeStruct((N, F), jnp.float32),
    )(agg1, agg2, n1, n2, ai, art, wfct, bfc)


def kernel(h, edge_index1, edge_index2, W_lin, b_lin, al, ar, W_fc, b_fc):
    ei1 = edge_index1.reshape(-1)
    ei2 = edge_index2.reshape(-1)

    zeros_deg = jnp.zeros((DEG_N,), jnp.float32)
    deg1_flat, deg2_flat = _make_degree_kernel()(ei1, ei2, zeros_deg)
    deg1 = deg1_flat.reshape(DEG_N, 1)
    deg2 = deg2_flat.reshape(DEG_N, 1)

    wlt = W_lin.T
    bl = b_lin.reshape(1, F)
    alt = jnp.tile(al[:, :, 0], (1, NUM_HEADS))
    art = jnp.tile(ar[:, :, 0], (1, NUM_HEADS))

    t1, t2, n1, n2, ai = _tc_prep(h, wlt, bl, deg1, deg2, alt)

    zeros_rows = jnp.zeros((80, F), jnp.float32)
    agg1, agg2 = _make_propagate_kernel()(t1, t2, ei1, ei2, zeros_rows)

    return _tc_combine(agg1, agg2, n1, n2, ai, art, W_fc.T,
                       b_fc.reshape(1, F))
